# baseline (speedup 1.0000x reference)
import functools

import jax
import jax.numpy as jnp
from jax import lax
from jax.experimental import pallas as pl
from jax.experimental.pallas import tpu as pltpu
from jax.experimental.pallas import tpu_sc as plsc


def _make_gather(B: int, S: int, V: int, D: int):
    NW = 32
    N = B * S
    per_w = N // NW
    w_per_row = S // per_w
    CH = 16
    GPH = 4
    n_ch = per_w // CH
    n_grp = n_ch // GPH

    mesh = plsc.VectorSubcoreMesh(core_axis_name="c", subcore_axis_name="s")

    @functools.partial(
        pl.kernel,
        mesh=mesh,
        out_type=jax.ShapeDtypeStruct((B, S, D), jnp.float32),
        scratch_types=[
            pltpu.VMEM((per_w,), jnp.int32),
            pltpu.VMEM((2 * GPH * CH, D), jnp.float32),
            pltpu.SemaphoreType.DMA((2 * GPH,)),
            pltpu.SemaphoreType.DMA((2,)),
        ],
    )
    def gather_kernel(idx_hbm, table_hbm, out_hbm, idx_v, bufs, gsem, ssem):
        wid = lax.axis_index("s") * 2 + lax.axis_index("c")
        b = wid // w_per_row
        col0 = (wid % w_per_row) * per_w
        pltpu.sync_copy(idx_hbm.at[b, pl.ds(col0, per_w)], idx_v)

        def start_gathers(g, h):
            for s in range(GPH):
                slot = h * GPH + s
                pltpu.async_copy(
                    table_hbm.at[idx_v.at[pl.ds((g * GPH + s) * CH, CH)]],
                    bufs.at[pl.ds(slot * CH, CH)], gsem.at[slot])

        def wait_gathers(h):
            for s in range(GPH):
                slot = h * GPH + s
                pltpu.make_async_copy(
                    table_hbm.at[pl.ds(0, CH)],
                    bufs.at[pl.ds(slot * CH, CH)], gsem.at[slot]).wait()

        def start_store(g, h):
            pltpu.async_copy(
                bufs.at[pl.ds(h * GPH * CH, GPH * CH)],
                out_hbm.at[b, pl.ds(col0 + g * GPH * CH, GPH * CH)],
                ssem.at[h])

        def wait_store(h):
            pltpu.make_async_copy(
                bufs.at[pl.ds(h * GPH * CH, GPH * CH)],
                out_hbm.at[b, pl.ds(col0, GPH * CH)], ssem.at[h]).wait()

        start_gathers(0, 0)
        wait_gathers(0)
        start_store(0, 0)
        start_gathers(1, 1)

        def body(k, _):
            g1 = 2 * k + 1
            wait_gathers(1)
            start_store(g1, 1)
            wait_store(0)
            start_gathers(g1 + 1, 0)
            wait_gathers(0)
            start_store(g1 + 1, 0)
            wait_store(1)
            start_gathers(g1 + 2, 1)
            return _
        lax.fori_loop(0, (n_grp - 2) // 2, body, 0)

        wait_gathers(1)
        start_store(n_grp - 1, 1)
        wait_store(0)
        wait_store(1)

    return gather_kernel


def kernel(input_ids, table):
    B, S = input_ids.shape
    V, D = table.shape
    return _make_gather(B, S, V, D)(input_ids, table)


# speedup vs baseline: 1.0166x; 1.0166x over previous
import functools

import jax
import jax.numpy as jnp
from jax import lax
from jax.experimental import pallas as pl
from jax.experimental.pallas import tpu as pltpu
from jax.experimental.pallas import tpu_sc as plsc


def _make_gather(B: int, S: int, V: int, D: int):
    NW = 32
    N = B * S
    per_w = N // NW
    w_per_row = S // per_w
    CH = 16
    NBUF = 8
    DEPTH = 6
    n_ch = per_w // CH

    mesh = plsc.VectorSubcoreMesh(core_axis_name="c", subcore_axis_name="s")

    @functools.partial(
        pl.kernel,
        mesh=mesh,
        out_type=jax.ShapeDtypeStruct((B, S, D), jnp.float32),
        scratch_types=(
            [pltpu.VMEM((per_w,), jnp.int32),
             pltpu.VMEM((NBUF * CH, D), jnp.float32),
             pltpu.SemaphoreType.DMA((NBUF,)),
             pltpu.SemaphoreType.DMA((NBUF,))]
        ),
    )
    def gather_kernel(idx_hbm, table_hbm, out_hbm, idx_v, bufs, gsem_a, ssem_a):
        rows = [bufs.at[pl.ds(s * CH, CH)] for s in range(NBUF)]
        gsem = [gsem_a.at[s] for s in range(NBUF)]
        ssem = [ssem_a.at[s] for s in range(NBUF)]
        wid = lax.axis_index("s") * 2 + lax.axis_index("c")
        b = wid // w_per_row
        col0 = (wid % w_per_row) * per_w
        pltpu.sync_copy(idx_hbm.at[b, pl.ds(col0, per_w)], idx_v)

        def start_gather(i, slot):
            return pltpu.async_copy(
                table_hbm.at[idx_v.at[pl.ds(i * CH, CH)]],
                rows[slot], gsem[slot])

        def start_store(i, slot):
            return pltpu.async_copy(
                rows[slot], out_hbm.at[b, pl.ds(col0 + i * CH, CH)],
                ssem[slot])

        def wait_gather(slot):
            pltpu.make_async_copy(
                table_hbm.at[pl.ds(0, CH)], rows[slot], gsem[slot]).wait()

        def wait_store(slot):
            pltpu.make_async_copy(
                rows[slot], out_hbm.at[b, pl.ds(col0, CH)], ssem[slot]).wait()

        def group(k, first, last):
            for s in range(NBUF):
                i = k * NBUF + s
                wait_gather(s)
                start_store(i, s)
                jslot = (s + DEPTH) % NBUF
                if (not last) or s < NBUF - DEPTH:
                    if not (first and s < NBUF - DEPTH):
                        wait_store(jslot)
                    start_gather(i + DEPTH, jslot)

        n_grp = n_ch // NBUF
        for s in range(DEPTH):
            start_gather(s, s)
        group(0, True, False)

        def body(k, _):
            group(k, False, False)
            return _
        lax.fori_loop(1, n_grp - 1, body, 0)

        group(n_grp - 1, False, True)
        for s in range(NBUF):
            wait_store(s)

    return gather_kernel


def kernel(input_ids, table):
    B, S = input_ids.shape
    V, D = table.shape
    return _make_gather(B, S, V, D)(input_ids, table)
